# baseline (speedup 1.0000x reference)
import jax
import jax.numpy as jnp
from jax import lax
from jax.experimental import pallas as pl
from jax.experimental.pallas import tpu as pltpu
from jax.experimental.pallas import tpu_sc as plsc

NC, NS = 2, 16
NW = NC * NS
CHUNK = 64
NBUF = 10
GAHEAD = 7


def _make_gather(total: int, dim: int):
    ids_per_w = total // NW
    nchunk = ids_per_w // CHUNK
    nouter = nchunk // NBUF

    def body(table_hbm, idx_hbm, out_hbm, idx_v, rows_v, gsem, osem):
        wid = lax.axis_index("s") * NC + lax.axis_index("c")
        base = wid * ids_per_w

        def out_at(ci):
            return out_hbm.at[pl.ds(base + ci * CHUNK, CHUNK)]

        pltpu.sync_copy(idx_hbm.at[pl.ds(base, ids_per_w)], idx_v)

        def idx_at(ci):
            return idx_v.at[pl.ds(ci * CHUNK, CHUNK)]

        def start_gather(ci, buf):
            pltpu.async_copy(table_hbm.at[idx_at(ci)], rows_v.at[buf], gsem)

        for b in range(GAHEAD):
            start_gather(b, b)

        @pl.loop(0, nouter)
        def _(gi):
            for b in range(NBUF):
                ci = gi * NBUF + b
                nb = (b + GAHEAD) % NBUF

                def wait_store(pci=ci + GAHEAD - NBUF, pb=nb):
                    pltpu.make_async_copy(rows_v.at[pb], out_at(pci), osem).wait()

                if b < NBUF - GAHEAD:
                    @pl.when(gi > 0)
                    def _():
                        wait_store()
                else:
                    wait_store()

                if b < NBUF - GAHEAD:
                    start_gather(ci + GAHEAD, nb)
                else:
                    @pl.when(gi < nouter - 1)
                    def _():
                        start_gather(ci + GAHEAD, nb)

                pltpu.make_async_copy(
                    table_hbm.at[idx_at(ci)], rows_v.at[b], gsem
                ).wait()
                pltpu.async_copy(rows_v.at[b], out_at(ci), osem)

        for b in range(GAHEAD, NBUF):
            ci = (nouter - 1) * NBUF + b
            pltpu.make_async_copy(rows_v.at[b], out_at(ci), osem).wait()

    return pl.kernel(
        body,
        out_type=jax.ShapeDtypeStruct((total, dim), jnp.float32),
        mesh=plsc.VectorSubcoreMesh(core_axis_name="c", subcore_axis_name="s"),
        scratch_types=[
            pltpu.VMEM((ids_per_w,), jnp.int32),
            pltpu.VMEM((NBUF, CHUNK, dim), jnp.float32),
            pltpu.SemaphoreType.DMA,
            pltpu.SemaphoreType.DMA,
        ],
    )


def kernel(annotations, ids):
    batch, hist = ids.shape
    vocab, dim = annotations.shape
    total = batch * hist
    flat = ids.reshape(total).astype(jnp.int32)

    grain = NW * CHUNK * NBUF
    padded = (total + grain - 1) // grain * grain
    if padded != total:
        flat = jnp.pad(flat, (0, padded - total))

    out = _make_gather(padded, dim)(annotations, flat)
    return out[:total].reshape(batch, hist, dim)


# speedup vs baseline: 1.7909x; 1.7909x over previous
import jax
import jax.numpy as jnp
from jax import lax
from jax.experimental import pallas as pl
from jax.experimental.pallas import tpu as pltpu
from jax.experimental.pallas import tpu_sc as plsc

NC, NS = 2, 16
NW = NC * NS
NBUF = 8
GAHEAD = 6


def _make_row_gather(batch: int, hist: int, dim: int):
    rows_per_w = batch // NW
    nouter = rows_per_w // NBUF

    def body(table_hbm, ids_hbm, out_hbm, idx_v, rows_v, gsem, osem):
        wid = lax.axis_index("s") * NC + lax.axis_index("c")
        base = wid * rows_per_w

        pltpu.sync_copy(ids_hbm.at[pl.ds(base, rows_per_w), :], idx_v)

        def start_gather(r, buf):
            pltpu.async_copy(table_hbm.at[idx_v.at[r]], rows_v.at[buf], gsem)

        for b in range(GAHEAD):
            start_gather(b, b)

        @pl.loop(0, nouter)
        def _(gi):
            for b in range(NBUF):
                r = gi * NBUF + b
                nb = (b + GAHEAD) % NBUF

                def wait_store(pr=r + GAHEAD - NBUF, pb=nb):
                    pltpu.make_async_copy(
                        rows_v.at[pb], out_hbm.at[base + pr], osem
                    ).wait()

                if b < NBUF - GAHEAD:
                    @pl.when(gi > 0)
                    def _():
                        wait_store()
                else:
                    wait_store()

                if b < NBUF - GAHEAD:
                    start_gather(r + GAHEAD, nb)
                else:
                    @pl.when(gi < nouter - 1)
                    def _():
                        start_gather(r + GAHEAD, nb)

                pltpu.make_async_copy(
                    table_hbm.at[idx_v.at[r]], rows_v.at[b], gsem
                ).wait()
                pltpu.async_copy(rows_v.at[b], out_hbm.at[base + r], osem)

        for b in range(GAHEAD, NBUF):
            r = (nouter - 1) * NBUF + b
            pltpu.make_async_copy(rows_v.at[b], out_hbm.at[base + r], osem).wait()

    return pl.kernel(
        body,
        out_type=jax.ShapeDtypeStruct((batch, hist, dim), jnp.float32),
        mesh=plsc.VectorSubcoreMesh(core_axis_name="c", subcore_axis_name="s"),
        scratch_types=[
            pltpu.VMEM((rows_per_w, hist), jnp.int32),
            pltpu.VMEM((NBUF, hist, dim), jnp.float32),
            pltpu.SemaphoreType.DMA,
            pltpu.SemaphoreType.DMA,
        ],
    )


F_CHUNK = 128
F_NBUF = 5
F_GAHEAD = 3


def _make_flat_gather(total: int, dim: int):
    ids_per_w = total // NW
    nchunk = ids_per_w // F_CHUNK
    nouter = nchunk // F_NBUF

    def body(table_hbm, idx_hbm, out_hbm, idx_v, rows_v, gsem, osem):
        wid = lax.axis_index("s") * NC + lax.axis_index("c")
        base = wid * ids_per_w

        def out_at(ci):
            return out_hbm.at[pl.ds(base + ci * F_CHUNK, F_CHUNK)]

        pltpu.sync_copy(idx_hbm.at[pl.ds(base, ids_per_w)], idx_v)

        def idx_at(ci):
            return idx_v.at[pl.ds(ci * F_CHUNK, F_CHUNK)]

        def start_gather(ci, buf):
            pltpu.async_copy(table_hbm.at[idx_at(ci)], rows_v.at[buf], gsem)

        for b in range(F_GAHEAD):
            start_gather(b, b)

        @pl.loop(0, nouter)
        def _(gi):
            for b in range(F_NBUF):
                ci = gi * F_NBUF + b
                nb = (b + F_GAHEAD) % F_NBUF

                def wait_store(pci=ci + F_GAHEAD - F_NBUF, pb=nb):
                    pltpu.make_async_copy(rows_v.at[pb], out_at(pci), osem).wait()

                if b < F_NBUF - F_GAHEAD:
                    @pl.when(gi > 0)
                    def _():
                        wait_store()
                else:
                    wait_store()

                if b < F_NBUF - F_GAHEAD:
                    start_gather(ci + F_GAHEAD, nb)
                else:
                    @pl.when(gi < nouter - 1)
                    def _():
                        start_gather(ci + F_GAHEAD, nb)

                pltpu.make_async_copy(
                    table_hbm.at[idx_at(ci)], rows_v.at[b], gsem
                ).wait()
                pltpu.async_copy(rows_v.at[b], out_at(ci), osem)

        for b in range(F_GAHEAD, F_NBUF):
            ci = (nouter - 1) * F_NBUF + b
            pltpu.make_async_copy(rows_v.at[b], out_at(ci), osem).wait()

    return pl.kernel(
        body,
        out_type=jax.ShapeDtypeStruct((total, dim), jnp.float32),
        mesh=plsc.VectorSubcoreMesh(core_axis_name="c", subcore_axis_name="s"),
        scratch_types=[
            pltpu.VMEM((ids_per_w,), jnp.int32),
            pltpu.VMEM((F_NBUF, F_CHUNK, dim), jnp.float32),
            pltpu.SemaphoreType.DMA,
            pltpu.SemaphoreType.DMA,
        ],
    )


def kernel(annotations, ids):
    batch, hist = ids.shape
    vocab, dim = annotations.shape
    ids = ids.astype(jnp.int32)

    if batch % (NW * NBUF) == 0 and hist <= 128:
        return _make_row_gather(batch, hist, dim)(annotations, ids)

    total = batch * hist
    flat = ids.reshape(total)
    grain = NW * F_CHUNK * F_NBUF
    padded = (total + grain - 1) // grain * grain
    if padded != total:
        flat = jnp.pad(flat, (0, padded - total))
    out = _make_flat_gather(padded, dim)(annotations, flat)
    return out[:total].reshape(batch, hist, dim)


# speedup vs baseline: 3.1669x; 1.7683x over previous
import jax
import jax.numpy as jnp
from jax import lax
from jax.experimental import pallas as pl
from jax.experimental.pallas import tpu as pltpu
from jax.experimental.pallas import tpu_sc as plsc

NC, NS = 2, 16
NW = NC * NS
CHUNK = 128
NBUF = 5
GAHEAD = 3


def _make_plane_gather(batch: int, hist: int, dim: int):
    nouter = hist // NBUF

    def body(table_hbm, idst_hbm, out_hbm, idx_v, rows_v, gsem, osem):
        wid = lax.axis_index("s") * NC + lax.axis_index("c")
        base = wid * CHUNK

        pltpu.sync_copy(idst_hbm.at[:, pl.ds(base, CHUNK)], idx_v)

        def start_gather(h, buf):
            pltpu.async_copy(table_hbm.at[idx_v.at[h]], rows_v.at[buf], gsem)

        for b in range(GAHEAD):
            start_gather(b, b)

        @pl.loop(0, nouter)
        def _(gi):
            for b in range(NBUF):
                h = gi * NBUF + b
                nb = (b + GAHEAD) % NBUF

                def wait_store(ph=h + GAHEAD - NBUF, pb=nb):
                    pltpu.make_async_copy(
                        rows_v.at[pb], out_hbm.at[ph, pl.ds(base, CHUNK)], osem
                    ).wait()

                if b < NBUF - GAHEAD:
                    @pl.when(gi > 0)
                    def _():
                        wait_store()
                else:
                    wait_store()

                if b < NBUF - GAHEAD:
                    start_gather(h + GAHEAD, nb)
                else:
                    @pl.when(gi < nouter - 1)
                    def _():
                        start_gather(h + GAHEAD, nb)

                pltpu.make_async_copy(
                    table_hbm.at[idx_v.at[h]], rows_v.at[b], gsem
                ).wait()
                pltpu.async_copy(
                    rows_v.at[b], out_hbm.at[h, pl.ds(base, CHUNK)], osem
                )

        for b in range(GAHEAD, NBUF):
            h = (nouter - 1) * NBUF + b
            pltpu.make_async_copy(
                rows_v.at[b], out_hbm.at[h, pl.ds(base, CHUNK)], osem
            ).wait()

    return pl.kernel(
        body,
        out_type=jax.ShapeDtypeStruct((hist, batch, dim), jnp.float32),
        mesh=plsc.VectorSubcoreMesh(core_axis_name="c", subcore_axis_name="s"),
        scratch_types=[
            pltpu.VMEM((hist, CHUNK), jnp.int32),
            pltpu.VMEM((NBUF, CHUNK, dim), jnp.float32),
            pltpu.SemaphoreType.DMA,
            pltpu.SemaphoreType.DMA,
        ],
    )


F_NBUF = 5
F_GAHEAD = 3


def _make_flat_gather(total: int, dim: int):
    ids_per_w = total // NW
    nchunk = ids_per_w // CHUNK
    nouter = nchunk // F_NBUF

    def body(table_hbm, idx_hbm, out_hbm, idx_v, rows_v, gsem, osem):
        wid = lax.axis_index("s") * NC + lax.axis_index("c")
        base = wid * ids_per_w

        def out_at(ci):
            return out_hbm.at[pl.ds(base + ci * CHUNK, CHUNK)]

        pltpu.sync_copy(idx_hbm.at[pl.ds(base, ids_per_w)], idx_v)

        def idx_at(ci):
            return idx_v.at[pl.ds(ci * CHUNK, CHUNK)]

        def start_gather(ci, buf):
            pltpu.async_copy(table_hbm.at[idx_at(ci)], rows_v.at[buf], gsem)

        for b in range(F_GAHEAD):
            start_gather(b, b)

        @pl.loop(0, nouter)
        def _(gi):
            for b in range(F_NBUF):
                ci = gi * F_NBUF + b
                nb = (b + F_GAHEAD) % F_NBUF

                def wait_store(pci=ci + F_GAHEAD - F_NBUF, pb=nb):
                    pltpu.make_async_copy(rows_v.at[pb], out_at(pci), osem).wait()

                if b < F_NBUF - F_GAHEAD:
                    @pl.when(gi > 0)
                    def _():
                        wait_store()
                else:
                    wait_store()

                if b < F_NBUF - F_GAHEAD:
                    start_gather(ci + F_GAHEAD, nb)
                else:
                    @pl.when(gi < nouter - 1)
                    def _():
                        start_gather(ci + F_GAHEAD, nb)

                pltpu.make_async_copy(
                    table_hbm.at[idx_at(ci)], rows_v.at[b], gsem
                ).wait()
                pltpu.async_copy(rows_v.at[b], out_at(ci), osem)

        for b in range(F_GAHEAD, F_NBUF):
            ci = (nouter - 1) * F_NBUF + b
            pltpu.make_async_copy(rows_v.at[b], out_at(ci), osem).wait()

    return pl.kernel(
        body,
        out_type=jax.ShapeDtypeStruct((total, dim), jnp.float32),
        mesh=plsc.VectorSubcoreMesh(core_axis_name="c", subcore_axis_name="s"),
        scratch_types=[
            pltpu.VMEM((ids_per_w,), jnp.int32),
            pltpu.VMEM((F_NBUF, CHUNK, dim), jnp.float32),
            pltpu.SemaphoreType.DMA,
            pltpu.SemaphoreType.DMA,
        ],
    )


def kernel(annotations, ids):
    batch, hist = ids.shape
    vocab, dim = annotations.shape
    ids = ids.astype(jnp.int32)

    if batch % (NW * CHUNK) == 0 and hist % NBUF == 0:
        ids_t = jnp.transpose(ids)
        out_t = _make_plane_gather(batch, hist, dim)(annotations, ids_t)
        return jnp.transpose(out_t, (1, 0, 2))

    total = batch * hist
    flat = ids.reshape(total)
    grain = NW * CHUNK * F_NBUF
    padded = (total + grain - 1) // grain * grain
    if padded != total:
        flat = jnp.pad(flat, (0, padded - total))
    out = _make_flat_gather(padded, dim)(annotations, flat)
    return out[:total].reshape(batch, hist, dim)
